# baseline (speedup 1.0000x reference)
import functools

import jax
import jax.numpy as jnp
from jax import lax
from jax.experimental import pallas as pl
from jax.experimental.pallas import tpu as pltpu
from jax.experimental.pallas import tpu_sc as plsc

B = 100000
V = 256
D = 128
K = 400
C = B // K
NC = 2
NS = 16
NW = NC * NS
NBUF = 2
CPW = C // NW
REM = C % NW
MAXC = CPW + 1


def kernel(state, batch_id):
    idx = batch_id.astype(jnp.int32)
    mesh = plsc.VectorSubcoreMesh(core_axis_name="c", subcore_axis_name="s")

    @functools.partial(
        pl.kernel,
        mesh=mesh,
        out_type=jax.ShapeDtypeStruct((B, D), jnp.float32),
        scratch_types=[
            pltpu.VMEM_SHARED((V, D), jnp.float32),
            pltpu.VMEM((MAXC * K,), jnp.int32),
            pltpu.VMEM((NBUF, K, D), jnp.float32),
            pltpu.SemaphoreType.DMA,
            pltpu.SemaphoreType.DMA((NBUF,)),
        ],
    )
    def run(state_hbm, idx_hbm, out_hbm, table_sh, idx_v, rows_v, gsem, osem):
        cid = lax.axis_index("c")
        sid = lax.axis_index("s")
        wid = sid * NC + cid

        @pl.when(sid == 0)
        def _():
            pltpu.sync_copy(state_hbm, table_sh)

        start = wid * CPW + jnp.minimum(wid, REM)
        n = CPW + jnp.where(wid < REM, 1, 0)
        pltpu.sync_copy(
            idx_hbm.at[pl.ds(start * K, CPW * K)], idx_v.at[pl.ds(0, CPW * K)]
        )

        @pl.when(wid < REM)
        def _():
            pltpu.sync_copy(
                idx_hbm.at[pl.ds((start + CPW) * K, K)],
                idx_v.at[pl.ds(CPW * K, K)],
            )

        plsc.subcore_barrier()

        def body(i, carry):
            b = i % NBUF

            @pl.when(i >= NBUF)
            def _():
                pltpu.make_async_copy(
                    rows_v.at[b], out_hbm.at[pl.ds(0, K)], osem.at[b]
                ).wait()

            pltpu.async_copy(
                table_sh.at[idx_v.at[pl.ds(i * K, K)]], rows_v.at[b], gsem
            ).wait()
            pltpu.async_copy(
                rows_v.at[b], out_hbm.at[pl.ds((start + i) * K, K)], osem.at[b]
            )
            return carry

        lax.fori_loop(0, n, body, 0)

        for b in range(NBUF):
            pltpu.make_async_copy(
                rows_v.at[b], out_hbm.at[pl.ds(0, K)], osem.at[b]
            ).wait()

    return run(state, idx)


# speedup vs baseline: 1.0024x; 1.0024x over previous
import functools

import jax
import jax.numpy as jnp
from jax import lax
from jax.experimental import pallas as pl
from jax.experimental.pallas import tpu as pltpu
from jax.experimental.pallas import tpu_sc as plsc

B = 100000
V = 256
D = 128
K = 200
C = B // K
NC = 2
NS = 16
NW = NC * NS
NBUF = 4
CPW = C // NW
REM = C % NW
MAXC = CPW + 1


def kernel(state, batch_id):
    idx = batch_id.astype(jnp.int32)
    mesh = plsc.VectorSubcoreMesh(core_axis_name="c", subcore_axis_name="s")

    @functools.partial(
        pl.kernel,
        mesh=mesh,
        out_type=jax.ShapeDtypeStruct((B, D), jnp.float32),
        scratch_types=[
            pltpu.VMEM_SHARED((V, D), jnp.float32),
            pltpu.VMEM((MAXC * K,), jnp.int32),
            pltpu.VMEM((NBUF, K, D), jnp.float32),
            pltpu.SemaphoreType.DMA,
            pltpu.SemaphoreType.DMA((NBUF,)),
        ],
    )
    def run(state_hbm, idx_hbm, out_hbm, table_sh, idx_v, rows_v, gsem, osem):
        cid = lax.axis_index("c")
        sid = lax.axis_index("s")
        wid = sid * NC + cid

        @pl.when(sid == 0)
        def _():
            pltpu.sync_copy(state_hbm, table_sh)

        start = wid * CPW + jnp.minimum(wid, REM)
        n = CPW + jnp.where(wid < REM, 1, 0)
        pltpu.sync_copy(
            idx_hbm.at[pl.ds(start * K, CPW * K)], idx_v.at[pl.ds(0, CPW * K)]
        )

        @pl.when(wid < REM)
        def _():
            pltpu.sync_copy(
                idx_hbm.at[pl.ds((start + CPW) * K, K)],
                idx_v.at[pl.ds(CPW * K, K)],
            )

        plsc.subcore_barrier()

        def body(i, carry):
            b = i % NBUF

            @pl.when(i >= NBUF)
            def _():
                pltpu.make_async_copy(
                    rows_v.at[b], out_hbm.at[pl.ds(0, K)], osem.at[b]
                ).wait()

            pltpu.async_copy(
                table_sh.at[idx_v.at[pl.ds(i * K, K)]], rows_v.at[b], gsem
            ).wait()
            pltpu.async_copy(
                rows_v.at[b], out_hbm.at[pl.ds((start + i) * K, K)], osem.at[b]
            )
            return carry

        lax.fori_loop(0, n, body, 0)

        for b in range(NBUF):
            pltpu.make_async_copy(
                rows_v.at[b], out_hbm.at[pl.ds(0, K)], osem.at[b]
            ).wait()

    return run(state, idx)
